# Initial kernel scaffold; baseline (speedup 1.0000x reference)
import jax
import jax.numpy as jnp
from jax.experimental import pallas as pl


def kernel(memb, demb, pemb, mirna_edgelist, mirna_edgeweight, disease_edge_list, disease_edgeweight, pcg_edge_list, pcg_edgeweight, mirna_pcg_pairs, disease_pcg_pairs, mirna_disease_pairs, Wm, bm, Wd, bd, Wp, bp, w_assoc, b_assoc, w_mp, b_mp, w_dp, b_dp):
    raise NotImplementedError("write your pallas kernel here")


# speedup vs baseline: 2.8833x; 2.8833x over previous
import jax
import jax.numpy as jnp
from jax.experimental import pallas as pl
from jax.experimental.pallas import tpu as pltpu

N = 10000
D = 128
H = 128
BN = 2000


def _mm_body(x_ref, w_ref, o_ref):
    o_ref[...] = jnp.dot(x_ref[0], w_ref[0],
                         preferred_element_type=jnp.float32)[None]


def _xw3(x3, w3):
    return pl.pallas_call(
        _mm_body,
        grid=(3, N // BN),
        in_specs=[
            pl.BlockSpec((1, BN, D), lambda g, i: (g, i, 0)),
            pl.BlockSpec((1, D, H), lambda g, i: (g, 0, 0)),
        ],
        out_specs=pl.BlockSpec((1, BN, H), lambda g, i: (g, i, 0)),
        out_shape=jax.ShapeDtypeStruct((3, N, H), jnp.float32),
    )(x3, w3)


def _gcn_rest(xw, edge_index, edge_weight, b):
    n = xw.shape[0]
    src = edge_index[:, 0]
    dst = edge_index[:, 1]
    ew = edge_weight
    deg = jax.ops.segment_sum(ew, dst, num_segments=n) + 1.0
    dis = jax.lax.rsqrt(deg)
    y = xw * dis[:, None]
    msg = y[src] * ew[:, None]
    acc = jax.ops.segment_sum(msg, dst, num_segments=n)
    return jax.nn.relu(dis[:, None] * (acc + y) + b)


def kernel(memb, demb, pemb, mirna_edgelist, mirna_edgeweight,
           disease_edge_list, disease_edgeweight, pcg_edge_list,
           pcg_edgeweight, mirna_pcg_pairs, disease_pcg_pairs,
           mirna_disease_pairs, Wm, bm, Wd, bd, Wp, bp, w_assoc, b_assoc,
           w_mp, b_mp, w_dp, b_dp):
    x3 = jnp.stack([memb, demb, pemb])
    w3 = jnp.stack([Wm, Wd, Wp])
    xw3 = _xw3(x3, w3)
    me = mirna_edgelist.astype(jnp.int32)
    de = disease_edge_list.astype(jnp.int32)
    pe = pcg_edge_list.astype(jnp.int32)
    mhid = _gcn_rest(xw3[0], me, mirna_edgeweight, bm)
    dhid = _gcn_rest(xw3[1], de, disease_edgeweight, bd)
    phid = _gcn_rest(xw3[2], pe, pcg_edgeweight, bp)

    mdp = mirna_disease_pairs.astype(jnp.int32)
    mpp = mirna_pcg_pairs.astype(jnp.int32)
    dpp = disease_pcg_pairs.astype(jnp.int32)
    assoc_vec = mhid[mdp[:, 0]] * dhid[mdp[:, 1]]
    mp_vec = mhid[mpp[:, 0]] * phid[mpp[:, 1]]
    dp_vec = dhid[dpp[:, 0]] * phid[dpp[:, 1]]
    assoc_out = jax.nn.sigmoid(assoc_vec @ w_assoc + b_assoc)[:, 0]
    mirna_pcg_out = jax.nn.sigmoid(mp_vec @ w_mp + b_mp)[:, 0]
    disease_pcg_out = jax.nn.sigmoid(dp_vec @ w_dp + b_dp)[:, 0]
    return (assoc_out, mirna_pcg_out, disease_pcg_out)


# speedup vs baseline: 6.8034x; 2.3596x over previous
import functools

import jax
import jax.numpy as jnp
from jax import lax
from jax.experimental import pallas as pl
from jax.experimental.pallas import tpu as pltpu
from jax.experimental.pallas import tpu_sc as plsc

N = 10000
D = 128
H = 128
E = 320000
P = 200000

NP = 10240
EP = 327680
PP = 204800
TOTP = 3 * PP

NSC = 2
NTL = 16
NW = NSC * NTL

ECH = EP // NW // 128
PCH = TOTP // NW // 128
NROW = NP // NTL

BN = 2048

_mesh = plsc.VectorSubcoreMesh(core_axis_name="c", subcore_axis_name="s")


def _deg_body(dstd, ewf, zeros_h, degp, deg_sh, dst_v, ew_v, zb):
    cid = lax.axis_index("c")
    sid = lax.axis_index("s")
    pltpu.sync_copy(zeros_h, zb)
    for z in range(3):
        pltpu.sync_copy(zb, deg_sh.at[pl.ds(sid * 1920 + z * 640, 640)])
    plsc.subcore_barrier()
    for g in range(3):
        rb = g * 2560 + cid * 1280 + sid * ECH
        pltpu.sync_copy(dstd.at[pl.ds(rb, ECH)], dst_v)
        pltpu.sync_copy(ewf.at[pl.ds(rb, ECH)], ew_v)

        def dchunk(ch, carry):
            pltpu.sync_copy(ew_v.at[ch], deg_sh.at[dst_v.at[ch]], add=True)
            return carry

        lax.fori_loop(0, ECH, dchunk, 0)
    plsc.subcore_barrier()
    for g in range(3):
        o = sid * 640
        pltpu.sync_copy(deg_sh.at[pl.ds(g * NP + o, 640)], zb)
        pltpu.sync_copy(zb,
                        degp.at[pl.ds(g * 2 * NP + cid * NP + o, 640)])


_deg_call = pl.kernel(
    _deg_body,
    out_type=jax.ShapeDtypeStruct((3 * 2 * NP,), jnp.float32),
    mesh=_mesh,
    scratch_types=[
        pltpu.VMEM_SHARED((3 * NP,), jnp.float32),
        pltpu.VMEM((ECH, 128), jnp.int32),
        pltpu.VMEM((ECH, 128), jnp.float32),
        pltpu.VMEM((640,), jnp.float32),
    ],
)


def _rsqrt_precise(d):
    r = lax.rsqrt(d)
    return r * (1.5 - 0.5 * d * r * r)


def _xwy_body(x_ref, w_ref, dg_ref, y_ref):
    xw = jnp.dot(x_ref[0], w_ref[0], preferred_element_type=jnp.float32)
    deg = dg_ref[0, 0] + dg_ref[0, 1] + 1.0
    dis = _rsqrt_precise(deg)
    y_ref[0] = xw * dis[:, None]


def _xwy(x3, w3, degp):
    return pl.pallas_call(
        _xwy_body,
        grid=(3, NP // BN),
        in_specs=[
            pl.BlockSpec((1, BN, D), lambda g, i: (g, i, 0)),
            pl.BlockSpec((1, D, H), lambda g, i: (g, 0, 0)),
            pl.BlockSpec((1, 2, BN), lambda g, i: (g, 0, i)),
        ],
        out_specs=pl.BlockSpec((1, BN, H), lambda g, i: (g, i, 0)),
        out_shape=jax.ShapeDtypeStruct((3, NP, H), jnp.float32),
    )(x3, w3, degp)


def _agg_body(srcf, dstf, ewf, yflat, zeros_h, accp,
              acc_sh, src_v, dst_v, ew_v, rows_v, sem):
    cid = lax.axis_index("c")
    sid = lax.axis_index("s")
    for g in range(3):
        pltpu.sync_copy(zeros_h, rows_v)
        for z in range(5):
            pltpu.sync_copy(rows_v, acc_sh.at[pl.ds(sid * NROW + z * 128, 128)])
        plsc.subcore_barrier()
        rb = g * 2560 + cid * 1280 + sid * ECH
        pltpu.sync_copy(srcf.at[pl.ds(rb, ECH)], src_v)
        pltpu.sync_copy(dstf.at[pl.ds(rb, ECH)], dst_v)
        pltpu.sync_copy(ewf.at[pl.ds(rb, ECH)], ew_v)

        def chunk(ch, carry):
            pltpu.async_copy(yflat.at[src_v.at[ch]], rows_v, sem).wait()

            def scale(kg, c2):
                ewv = ew_v[ch, pl.ds(kg * 16, 16)]
                for j in range(16):
                    k = kg * 16 + j
                    s = ewv[j]
                    for r in range(8):
                        rows_v[k, pl.ds(r * 16, 16)] = (
                            rows_v[k, pl.ds(r * 16, 16)] * s)
                return c2

            lax.fori_loop(0, 8, scale, 0)
            pltpu.sync_copy(rows_v, acc_sh.at[dst_v.at[ch]], add=True)
            return carry

        lax.fori_loop(0, ECH, chunk, 0)
        plsc.subcore_barrier()
        for z in range(5):
            o = sid * NROW + z * 128
            pltpu.sync_copy(acc_sh.at[pl.ds(o, 128)], rows_v)
            pltpu.sync_copy(
                rows_v, accp.at[pl.ds(g * 2 * NP + cid * NP + o, 128)])
        plsc.subcore_barrier()


_agg_call = pl.kernel(
    _agg_body,
    out_type=jax.ShapeDtypeStruct((3 * 2 * NP, H), jnp.float32),
    mesh=_mesh,
    scratch_types=[
        pltpu.VMEM_SHARED((NP, H), jnp.float32),
        pltpu.VMEM((ECH, 128), jnp.int32),
        pltpu.VMEM((ECH, 128), jnp.int32),
        pltpu.VMEM((ECH, 128), jnp.float32),
        pltpu.VMEM((128, H), jnp.float32),
        pltpu.SemaphoreType.DMA,
    ],
)


def _tab_body(acc0_ref, acc1_ref, y_ref, dg_ref, b_ref, t_ref):
    deg = dg_ref[0, 0] + dg_ref[0, 1] + 1.0
    dis = _rsqrt_precise(deg)
    pre = dis[:, None] * (acc0_ref[0] + acc1_ref[0] + y_ref[0])
    t_ref[0] = jnp.maximum(pre + b_ref[0][0], 0.0)


def _tables(accp, y3, degp, b3):
    return pl.pallas_call(
        _tab_body,
        grid=(3, NP // BN),
        in_specs=[
            pl.BlockSpec((1, BN, H), lambda g, i: (2 * g, i, 0)),
            pl.BlockSpec((1, BN, H), lambda g, i: (2 * g + 1, i, 0)),
            pl.BlockSpec((1, BN, H), lambda g, i: (g, i, 0)),
            pl.BlockSpec((1, 2, BN), lambda g, i: (g, 0, i)),
            pl.BlockSpec((1, 128, H), lambda g, i: (g, 0, 0)),
        ],
        out_specs=pl.BlockSpec((1, BN, H), lambda g, i: (g, i, 0)),
        out_shape=jax.ShapeDtypeStruct((3, NP, H), jnp.float32),
    )(accp, accp, y3, degp, b3)


def _pair_body(tflat, iaf, ibf, consts, wtab, dots,
               ia_v, ib_v, a_v, b_v, d_v, c_v, w_v, sem1, sem2):
    cid = lax.axis_index("c")
    sid = lax.axis_index("s")
    wid = cid * NTL + sid
    pltpu.sync_copy(consts, c_v)
    pltpu.sync_copy(wtab, w_v)
    rb = wid * PCH
    pltpu.sync_copy(iaf.at[pl.ds(rb * 128, PCH * 128)], ia_v)
    pltpu.sync_copy(ibf.at[pl.ds(rb * 128, PCH * 128)], ib_v)
    lanes = lax.iota(jnp.int32, 16)
    cvec = c_v[0, pl.ds(0, 16)]

    def chunk(ch, carry):
        d1 = pltpu.async_copy(
            tflat.at[ia_v.at[pl.ds(ch * 128, 128)]], a_v, sem1)
        d2 = pltpu.async_copy(
            tflat.at[ib_v.at[pl.ds(ch * 128, 128)]], b_v, sem2)
        d1.wait()
        d2.wait()
        s = ((rb + ch) * 128) // PP
        bias = jnp.where(s == 0, cvec[0], jnp.where(s == 1, cvec[1], cvec[2]))
        wv = [w_v[s, pl.ds(r * 16, 16)] for r in range(8)]

        def grp(kg, c2):
            dvec = jnp.zeros((16,), jnp.float32)
            for j in range(16):
                k = kg * 16 + j
                acc = a_v[k, pl.ds(0, 16)] * b_v[k, pl.ds(0, 16)] * wv[0]
                for r in range(1, 8):
                    acc = acc + (a_v[k, pl.ds(r * 16, 16)]
                                 * b_v[k, pl.ds(r * 16, 16)] * wv[r])
                d01 = (acc[0] + acc[1]) + (acc[2] + acc[3])
                d23 = (acc[4] + acc[5]) + (acc[6] + acc[7])
                d45 = (acc[8] + acc[9]) + (acc[10] + acc[11])
                d67 = (acc[12] + acc[13]) + (acc[14] + acc[15])
                dot = (d01 + d23) + (d45 + d67)
                dvec = jnp.where(lanes == j, dot, dvec)
            t = dvec + bias
            d_v[pl.ds(kg * 16, 16)] = 1.0 / (1.0 + jnp.exp(-t))
            return c2

        lax.fori_loop(0, 8, grp, 0)
        pltpu.sync_copy(d_v, dots.at[pl.ds((rb + ch) * 128, 128)])
        return carry

    lax.fori_loop(0, PCH, chunk, 0)


_pair_call = pl.kernel(
    _pair_body,
    out_type=jax.ShapeDtypeStruct((TOTP,), jnp.float32),
    mesh=_mesh,
    scratch_types=[
        pltpu.VMEM((PCH * 128,), jnp.int32),
        pltpu.VMEM((PCH * 128,), jnp.int32),
        pltpu.VMEM((128, H), jnp.float32),
        pltpu.VMEM((128, H), jnp.float32),
        pltpu.VMEM((128,), jnp.float32),
        pltpu.VMEM((1, 16), jnp.float32),
        pltpu.VMEM((3, 128), jnp.float32),
        pltpu.SemaphoreType.DMA,
        pltpu.SemaphoreType.DMA,
    ],
)


def _prep_edges(el, ew, g):
    src = el[:, 0].astype(jnp.int32) + g * NP
    dst = el[:, 1].astype(jnp.int32)
    pad_i = jnp.zeros((EP - E,), jnp.int32)
    src = jnp.concatenate([src, pad_i + g * NP])
    dstc = jnp.concatenate([dst, pad_i])
    ewp = jnp.concatenate([ew, jnp.zeros((EP - E,), jnp.float32)])
    return src, dstc, dstc + g * NP, ewp


def _prep_pairs(pr, offa, offb):
    ia = pr[:, 0].astype(jnp.int32) + offa
    ib = pr[:, 1].astype(jnp.int32) + offb
    pad = jnp.zeros((PP - P,), jnp.int32)
    return (jnp.concatenate([ia, pad + offa]),
            jnp.concatenate([ib, pad + offb]))


def kernel(memb, demb, pemb, mirna_edgelist, mirna_edgeweight,
           disease_edge_list, disease_edgeweight, pcg_edge_list,
           pcg_edgeweight, mirna_pcg_pairs, disease_pcg_pairs,
           mirna_disease_pairs, Wm, bm, Wd, bd, Wp, bp, w_assoc, b_assoc,
           w_mp, b_mp, w_dp, b_dp):
    f32 = jnp.float32
    pad_x = jnp.zeros((NP - N, D), f32)
    x3 = jnp.stack([jnp.concatenate([memb, pad_x]),
                    jnp.concatenate([demb, pad_x]),
                    jnp.concatenate([pemb, pad_x])])
    w3 = jnp.stack([Wm, Wd, Wp])

    eds = [_prep_edges(mirna_edgelist, mirna_edgeweight, 0),
           _prep_edges(disease_edge_list, disease_edgeweight, 1),
           _prep_edges(pcg_edge_list, pcg_edgeweight, 2)]
    srcf = jnp.concatenate([e[0] for e in eds]).reshape(3 * 2560, 128)
    dstf = jnp.concatenate([e[1] for e in eds]).reshape(3 * 2560, 128)
    dstd = jnp.concatenate([e[2] for e in eds]).reshape(3 * 2560, 128)
    ewf = jnp.concatenate([e[3] for e in eds]).reshape(3 * 2560, 128)

    zeros1 = jnp.zeros((640,), f32)
    zeros2 = jnp.zeros((128, H), f32)

    degp = _deg_call(dstd, ewf, zeros1).reshape(3, 2, NP)
    y3 = _xwy(x3, w3, degp)
    accp = _agg_call(srcf, dstf, ewf,
                     y3.reshape(3 * NP, H), zeros2).reshape(6, NP, H)

    b3 = jnp.broadcast_to(
        jnp.stack([bm, bd, bp])[:, None, :], (3, 128, H)).astype(f32)
    tabs = _tables(accp, y3, degp, b3)

    ia0, ib0 = _prep_pairs(mirna_disease_pairs, 0, NP)
    ia1, ib1 = _prep_pairs(mirna_pcg_pairs, 0, 2 * NP)
    ia2, ib2 = _prep_pairs(disease_pcg_pairs, NP, 2 * NP)
    iaf = jnp.concatenate([ia0, ia1, ia2])
    ibf = jnp.concatenate([ib0, ib1, ib2])
    consts = jnp.zeros((1, 16), f32)
    consts = consts.at[0, 0].set(b_assoc[0])
    consts = consts.at[0, 1].set(b_mp[0])
    consts = consts.at[0, 2].set(b_dp[0])
    wtab = jnp.stack([w_assoc[:, 0], w_mp[:, 0], w_dp[:, 0]]).astype(f32)

    sig = _pair_call(tabs.reshape(3 * NP, H), iaf, ibf, consts, wtab)
    return (sig[0:P], sig[PP:PP + P], sig[2 * PP:2 * PP + P])
